# baseline (speedup 1.0000x reference)
import functools

import jax
import jax.numpy as jnp
from jax import lax
from jax.experimental import pallas as pl
from jax.experimental.pallas import tpu as pltpu
from jax.experimental.pallas import tpu_sc as plsc

NC = 2
NS = 16
NW = NC * NS
CHUNK = 1024
NEWTON_ITERS = 2
STAGE_PTS = 6256

_MAGIC = jnp.int32(0x4B000000)
_SX = jnp.float32(1.0 / 128.0)
_SZ = jnp.float32(1.0 / 64.0)


def _newton_sqrt(x):
    xi = lax.bitcast_convert_type(x, jnp.int32)
    yi = jnp.int32(0x5F3759DF) - lax.shift_right_arithmetic(xi, 1)
    y = lax.bitcast_convert_type(yi, jnp.float32)
    half_x = 0.5 * x
    for _ in range(NEWTON_ITERS):
        y = y * (1.5 - half_x * y * y)
    return x * y


def _field_float(w, shift, mask):
    f = lax.shift_right_logical(w, shift) if shift else w
    if mask is not None:
        f = lax.bitwise_and(f, jnp.int32(mask))
    return lax.bitcast_convert_type(lax.bitwise_or(f, _MAGIC), jnp.float32)


def _make_kernel(E, N):
    nchunks = E // CHUNK
    assert nchunks * CHUNK == E
    ntrips_max = -(-nchunks // NW)
    stage_tail = N - (NS - 1) * STAGE_PTS
    assert 0 < stage_tail <= STAGE_PTS
    mesh = plsc.VectorSubcoreMesh(core_axis_name="c", subcore_axis_name="s")

    buf = lambda n, dt=jnp.float32: pltpu.VMEM((n,), dt)
    slot_types = [
        buf(CHUNK, jnp.int32),
        buf(CHUNK, jnp.int32),
        buf(CHUNK), buf(CHUNK), buf(CHUNK),
        buf(CHUNK, jnp.int32),
        buf(CHUNK, jnp.int32),
        buf(CHUNK),
    ]

    @functools.partial(
        pl.kernel,
        out_type=jax.ShapeDtypeStruct((E,), jnp.float32),
        mesh=mesh,
        scratch_types=slot_types + slot_types + [
            pltpu.SemaphoreType.DMA,
            pltpu.SemaphoreType.DMA,
            pltpu.SemaphoreType.DMA,
            pltpu.SemaphoreType.DMA,
            pltpu.SemaphoreType.DMA,
            pltpu.SemaphoreType.DMA,
            pltpu.VMEM_SHARED((N,), jnp.int32),
            buf(STAGE_PTS, jnp.int32),
        ],
        compiler_params=pltpu.CompilerParams(needs_layout_passes=False),
    )
    def kern(pt_hbm, ii_hbm, ij_hbm,
             ox_hbm, oy_hbm, oz_hbm, out_hbm, *rest):
        slots = (rest[0:8], rest[8:16])
        sem_ld = rest[16:18]
        sem_ga = rest[18:20]
        sem_out = rest[20:22]
        pt_sh = rest[22]
        st_v = rest[23]
        sid = lax.axis_index("s")
        wid = sid * NC + lax.axis_index("c")

        def stage(npts):
            sl = pl.ds(sid * STAGE_PTS, npts)
            sb = pl.ds(0, npts)
            pltpu.sync_copy(pt_hbm.at[sl], st_v.at[sb])
            pltpu.sync_copy(st_v.at[sb], pt_sh.at[sl])

        @pl.when(sid < NS - 1)
        def _():
            stage(STAGE_PTS)

        @pl.when(sid == NS - 1)
        def _():
            stage(stage_tail)

        plsc.subcore_barrier()

        def chunk_id(t):
            return wid + NW * t

        def issue_loads(k, p):
            ii_v, ij_v, ox_v, oy_v, oz_v = slots[p][0:5]
            base = k * CHUNK
            sl = pl.ds(base, CHUNK)
            pltpu.async_copy(ii_hbm.at[sl], ii_v, sem_ld[p])
            pltpu.async_copy(ij_hbm.at[sl], ij_v, sem_ld[p])
            pltpu.async_copy(ox_hbm.at[sl], ox_v, sem_ld[p])
            pltpu.async_copy(oy_hbm.at[sl], oy_v, sem_ld[p])
            pltpu.async_copy(oz_hbm.at[sl], oz_v, sem_ld[p])

        def wait_loads(p):
            for dst in slots[p][0:5]:
                pltpu.make_async_copy(out_hbm.at[pl.ds(0, CHUNK)], dst,
                                      sem_ld[p]).wait()

        def issue_gathers(p):
            ii_v, ij_v = slots[p][0:2]
            wi_v, wj_v = slots[p][5:7]
            pltpu.async_copy(pt_sh.at[ii_v], wi_v, sem_ga[p])
            pltpu.async_copy(pt_sh.at[ij_v], wj_v, sem_ga[p])

        def wait_gathers(p):
            for dst in slots[p][5:7]:
                pltpu.make_async_copy(out_hbm.at[pl.ds(0, CHUNK)],
                                      dst, sem_ga[p]).wait()

        def compute(k, p):
            (_, _, ox_v, oy_v, oz_v, wi_v, wj_v, out_v) = slots[p]
            for g in range(CHUNK // 16):
                sl = pl.ds(16 * g, 16)
                wi = wi_v[sl]
                wj = wj_v[sl]
                dx = (_field_float(wi, 21, None)
                      - _field_float(wj, 21, None)) * _SX - ox_v[sl]
                dy = (_field_float(wi, 10, 0x7FF)
                      - _field_float(wj, 10, 0x7FF)) * _SX - oy_v[sl]
                dz = (_field_float(wi, 0, 0x3FF)
                      - _field_float(wj, 0, 0x3FF)) * _SZ - oz_v[sl]
                acc = dx * dx + dy * dy + dz * dz
                out_v[sl] = _newton_sqrt(acc)
            pltpu.async_copy(out_v, out_hbm.at[pl.ds(k * CHUNK, CHUNK)],
                             sem_out[p])

        def wait_out(p):
            out_v = slots[p][7]
            pltpu.make_async_copy(out_hbm.at[pl.ds(0, CHUNK)], out_v,
                                  sem_out[p]).wait()

        issue_loads(chunk_id(0), 0)
        wait_loads(0)
        issue_gathers(0)

        def do_trip(t, p):
            k = chunk_id(t)
            knext = chunk_id(t + 1)
            nvalid = knext < nchunks

            @pl.when(nvalid)
            def _():
                issue_loads(knext, 1 - p)

            wait_gathers(p)

            @pl.when(nvalid)
            def _():
                wait_loads(1 - p)
                issue_gathers(1 - p)

            @pl.when(t >= 2)
            def _():
                wait_out(p)
            compute(k, p)

        def body(u, carry):
            t0 = u * 2

            @pl.when(chunk_id(t0) < nchunks)
            def _():
                do_trip(t0, 0)

            @pl.when(chunk_id(t0 + 1) < nchunks)
            def _():
                do_trip(t0 + 1, 1)
            return carry

        lax.fori_loop(0, (ntrips_max + 1) // 2, body, 0)
        pltpu.make_async_copy(out_hbm.at[pl.ds(0, CHUNK)], slots[0][7],
                              sem_out[0]).wait()
        pltpu.make_async_copy(out_hbm.at[pl.ds(0, CHUNK)], slots[1][7],
                              sem_out[1]).wait()

    return kern


def _quantize(v, scale, maxq):
    q = jnp.round((v + 8.0) * scale).astype(jnp.int32)
    return jnp.clip(q, 0, maxq)


def kernel(r, offsets, idx_ik, idx_jk):
    B, N, _ = r.shape
    E = idx_ik.shape[1]
    qx = _quantize(r[0, :, 0], 128.0, 2047)
    qy = _quantize(r[0, :, 1], 128.0, 2047)
    qz = _quantize(r[0, :, 2], 64.0, 1023)
    pt = lax.bitwise_or(
        lax.bitwise_or(lax.shift_left(qx, 21), lax.shift_left(qy, 10)), qz)
    out = _make_kernel(E, N)(pt,
                             idx_ik[0], idx_jk[0],
                             offsets[0, :, 0], offsets[0, :, 1],
                             offsets[0, :, 2])
    return out.reshape(B, E, 1)


# speedup vs baseline: 1.4693x; 1.4693x over previous
import functools

import jax
import jax.numpy as jnp
from jax import lax
from jax.experimental import pallas as pl
from jax.experimental.pallas import tpu as pltpu
from jax.experimental.pallas import tpu_sc as plsc

NC = 2
NS = 16
NW = NC * NS
CHUNK = 1024
NEWTON_ITERS = 1
STAGE_PTS = 6256


def _newton_sqrt(x):
    xi = lax.bitcast_convert_type(x, jnp.int32)
    yi = jnp.int32(0x5F3759DF) - lax.shift_right_arithmetic(xi, 1)
    y = lax.bitcast_convert_type(yi, jnp.float32)
    half_x = 0.5 * x
    for _ in range(NEWTON_ITERS):
        y = y * (1.5 - half_x * y * y)
    return x * y


def _unpack_xy(w):
    x = lax.bitcast_convert_type(
        lax.bitwise_and(w, jnp.int32(-65536)), jnp.float32)
    y = lax.bitcast_convert_type(lax.shift_left(w, 16), jnp.float32)
    return x, y


def _make_kernel(E, N):
    nchunks = E // CHUNK
    assert nchunks * CHUNK == E
    ntrips_max = -(-nchunks // NW)
    stage_tail = N - (NS - 1) * STAGE_PTS
    assert 0 < stage_tail <= STAGE_PTS
    mesh = plsc.VectorSubcoreMesh(core_axis_name="c", subcore_axis_name="s")

    buf = lambda n, dt=jnp.float32: pltpu.VMEM((n,), dt)
    slot_types = [
        buf(CHUNK, jnp.int32),
        buf(CHUNK, jnp.int32),
        buf(CHUNK), buf(CHUNK), buf(CHUNK),
        buf(CHUNK, jnp.int32),
        buf(CHUNK),
        buf(CHUNK, jnp.int32),
        buf(CHUNK),
        buf(CHUNK),
    ]

    @functools.partial(
        pl.kernel,
        out_type=jax.ShapeDtypeStruct((E,), jnp.float32),
        mesh=mesh,
        scratch_types=slot_types + slot_types + [
            pltpu.SemaphoreType.DMA,
            pltpu.SemaphoreType.DMA,
            pltpu.SemaphoreType.DMA,
            pltpu.SemaphoreType.DMA,
            pltpu.SemaphoreType.DMA,
            pltpu.SemaphoreType.DMA,
            pltpu.VMEM_SHARED((N,), jnp.int32),
            pltpu.VMEM_SHARED((N,), jnp.float32),
            buf(STAGE_PTS, jnp.int32),
            buf(STAGE_PTS),
        ],
        compiler_params=pltpu.CompilerParams(needs_layout_passes=False),
    )
    def kern(xy_hbm, rz_hbm, ii_hbm, ij_hbm,
             ox_hbm, oy_hbm, oz_hbm, out_hbm, *rest):
        slots = (rest[0:10], rest[10:20])
        sem_ld = rest[20:22]
        sem_ga = rest[22:24]
        sem_out = rest[24:26]
        xy_sh, rz_sh = rest[26:28]
        sti_v, stf_v = rest[28:30]
        sid = lax.axis_index("s")
        wid = sid * NC + lax.axis_index("c")

        def stage(npts):
            sl = pl.ds(sid * STAGE_PTS, npts)
            sb = pl.ds(0, npts)
            pltpu.sync_copy(xy_hbm.at[sl], sti_v.at[sb])
            pltpu.sync_copy(sti_v.at[sb], xy_sh.at[sl])
            pltpu.sync_copy(rz_hbm.at[sl], stf_v.at[sb])
            pltpu.sync_copy(stf_v.at[sb], rz_sh.at[sl])

        @pl.when(sid < NS - 1)
        def _():
            stage(STAGE_PTS)

        @pl.when(sid == NS - 1)
        def _():
            stage(stage_tail)

        plsc.subcore_barrier()

        def chunk_id(t):
            return wid + NW * t

        def issue_loads(k, p):
            ii_v, ij_v, ox_v, oy_v, oz_v = slots[p][0:5]
            base = k * CHUNK
            sl = pl.ds(base, CHUNK)
            pltpu.async_copy(ii_hbm.at[sl], ii_v, sem_ld[p])
            pltpu.async_copy(ij_hbm.at[sl], ij_v, sem_ld[p])
            pltpu.async_copy(ox_hbm.at[sl], ox_v, sem_ld[p])
            pltpu.async_copy(oy_hbm.at[sl], oy_v, sem_ld[p])
            pltpu.async_copy(oz_hbm.at[sl], oz_v, sem_ld[p])

        def wait_loads(p):
            for dst in slots[p][0:5]:
                pltpu.make_async_copy(out_hbm.at[pl.ds(0, CHUNK)], dst,
                                      sem_ld[p]).wait()

        def issue_gathers(p):
            (ii_v, ij_v, _ox, _oy, _oz,
             wi_v, zi_v, wj_v, zj_v, _o) = slots[p]
            for tab, idx_v, dst in ((xy_sh, ii_v, wi_v),
                                    (rz_sh, ii_v, zi_v),
                                    (xy_sh, ij_v, wj_v),
                                    (rz_sh, ij_v, zj_v)):
                pltpu.async_copy(tab.at[idx_v], dst, sem_ga[p])

        def wait_gathers(p):
            for dst in slots[p][5:9]:
                pltpu.make_async_copy(out_hbm.at[pl.ds(0, CHUNK)],
                                      dst, sem_ga[p]).wait()

        def compute(k, p):
            (_, _, ox_v, oy_v, oz_v,
             wi_v, zi_v, wj_v, zj_v, out_v) = slots[p]
            for g in range(CHUNK // 16):
                sl = pl.ds(16 * g, 16)
                xi, yi = _unpack_xy(wi_v[sl])
                xj, yj = _unpack_xy(wj_v[sl])
                dx = xi - xj - ox_v[sl]
                dy = yi - yj - oy_v[sl]
                dz = zi_v[sl] - zj_v[sl] - oz_v[sl]
                acc = dx * dx + dy * dy + dz * dz
                out_v[sl] = _newton_sqrt(acc)
            pltpu.async_copy(out_v, out_hbm.at[pl.ds(k * CHUNK, CHUNK)],
                             sem_out[p])

        def wait_out(p):
            out_v = slots[p][9]
            pltpu.make_async_copy(out_hbm.at[pl.ds(0, CHUNK)], out_v,
                                  sem_out[p]).wait()

        issue_loads(chunk_id(0), 0)
        wait_loads(0)
        issue_gathers(0)

        def do_trip(t, p):
            k = chunk_id(t)
            knext = chunk_id(t + 1)
            nvalid = knext < nchunks

            @pl.when(nvalid)
            def _():
                issue_loads(knext, 1 - p)

            wait_gathers(p)

            @pl.when(nvalid)
            def _():
                wait_loads(1 - p)
                issue_gathers(1 - p)

            @pl.when(t >= 2)
            def _():
                wait_out(p)
            compute(k, p)

        def body(u, carry):
            t0 = u * 2

            @pl.when(chunk_id(t0) < nchunks)
            def _():
                do_trip(t0, 0)

            @pl.when(chunk_id(t0 + 1) < nchunks)
            def _():
                do_trip(t0 + 1, 1)
            return carry

        lax.fori_loop(0, (ntrips_max + 1) // 2, body, 0)
        pltpu.make_async_copy(out_hbm.at[pl.ds(0, CHUNK)], slots[0][9],
                              sem_out[0]).wait()
        pltpu.make_async_copy(out_hbm.at[pl.ds(0, CHUNK)], slots[1][9],
                              sem_out[1]).wait()

    return kern


def kernel(r, offsets, idx_ik, idx_jk):
    B, N, _ = r.shape
    E = idx_ik.shape[1]
    xb = lax.bitcast_convert_type(
        r[0, :, 0].astype(jnp.bfloat16), jnp.uint16).astype(jnp.uint32)
    yb = lax.bitcast_convert_type(
        r[0, :, 1].astype(jnp.bfloat16), jnp.uint16).astype(jnp.uint32)
    xy = lax.bitcast_convert_type(
        lax.bitwise_or(lax.shift_left(xb, jnp.uint32(16)), yb), jnp.int32)
    out = _make_kernel(E, N)(xy, r[0, :, 2],
                             idx_ik[0], idx_jk[0],
                             offsets[0, :, 0], offsets[0, :, 1],
                             offsets[0, :, 2])
    return out.reshape(B, E, 1)
